# Initial kernel scaffold; baseline (speedup 1.0000x reference)
import jax
import jax.numpy as jnp
from jax.experimental import pallas as pl


def kernel(batch, edge_index, edge_type, query_table, rel0, W0, b0, rel1, W1, b1, mlp_W0, mlp_b0, mlp_W1, mlp_b1):
    raise NotImplementedError("write your pallas kernel here")


# speedup vs baseline: 22.2677x; 22.2677x over previous
import functools

import jax
import jax.numpy as jnp
from jax import lax
from jax.experimental import pallas as pl
from jax.experimental.pallas import tpu as pltpu
from jax.experimental.pallas import tpu_sc as plsc

_N = 10000
_E = 160000
_R = 32
_D = 128
_B = 2
_NEG = 16

_CH = 128
_EPT = 10112
_EPAD = _EPT * 16
_NCHUNK = _EPT // _CH
_NACC = 10240
_ZROWS = _NACC // 16

_mesh = plsc.VectorSubcoreMesh(core_axis_name="c", subcore_axis_name="s")


@functools.partial(
    pl.kernel,
    mesh=_mesh,
    out_type=jax.ShapeDtypeStruct((_B * _N, _D), jnp.float32),
    scratch_types=[
        pltpu.VMEM((_CH,), jnp.int32),
        pltpu.VMEM((_CH,), jnp.int32),
        pltpu.VMEM((_CH,), jnp.int32),
        pltpu.VMEM((_CH, _D), jnp.float32),
        pltpu.VMEM((_R, _D), jnp.float32),
        pltpu.VMEM_SHARED((_NACC, _D), jnp.float32),
        pltpu.SemaphoreType.DMA,
    ],
)
def _mp_kernel(state, src, dst, et, rel, out, srcv, dstv, etv, rows, relv,
               acc, sem):
    c = lax.axis_index("c")
    s = lax.axis_index("s")

    pltpu.sync_copy(rel, relv)

    zero16 = jnp.zeros((16,), jnp.float32)

    def _zrow(i, carry):
        for j in range(_D // 16):
            rows[i, pl.ds(j * 16, 16)] = zero16
        return carry

    lax.fori_loop(0, _CH, _zrow, 0)
    zbase = s * _ZROWS
    for k in range(_ZROWS // _CH):
        pltpu.sync_copy(rows, acc.at[pl.ds(zbase + k * _CH, _CH)])
    plsc.subcore_barrier()

    ebase = s * _EPT
    shift = c * _N

    def _chunk(k, carry):
        off = ebase + k * _CH
        pltpu.sync_copy(src.at[pl.ds(off, _CH)], srcv)
        pltpu.sync_copy(dst.at[pl.ds(off, _CH)], dstv)
        pltpu.sync_copy(et.at[pl.ds(off, _CH)], etv)
        for v in range(_CH // 16):
            srcv[pl.ds(v * 16, 16)] = srcv[pl.ds(v * 16, 16)] + shift
        pltpu.async_copy(state.at[srcv], rows, sem).wait()

        def _group(g, ecarry):
            et_vec = etv[pl.ds(g * 16, 16)]
            for l in range(16):
                et_i = et_vec[l]
                i = g * 16 + l
                for j in range(_D // 16):
                    rows[i, pl.ds(j * 16, 16)] = (
                        rows[i, pl.ds(j * 16, 16)]
                        * relv[et_i, pl.ds(j * 16, 16)])
            return ecarry

        lax.fori_loop(0, _CH // 16, _group, 0)
        pltpu.sync_copy(rows, acc.at[dstv], add=True)
        return carry

    lax.fori_loop(0, _NCHUNK, _chunk, 0)
    plsc.subcore_barrier()

    pltpu.sync_copy(acc.at[pl.ds(s * 624, 624)],
                    out.at[pl.ds(c * _N + s * 624, 624)])

    @pl.when(s == 0)
    def _():
        pltpu.sync_copy(acc.at[pl.ds(9984, 16)],
                        out.at[pl.ds(c * _N + 9984, 16)])


@functools.partial(
    pl.kernel,
    mesh=_mesh,
    out_type=jax.ShapeDtypeStruct((_B * _NEG, _D), jnp.float32),
    scratch_types=[
        pltpu.VMEM((_NEG,), jnp.int32),
        pltpu.VMEM((_NEG, _D), jnp.float32),
        pltpu.SemaphoreType.DMA,
    ],
)
def _tail_gather_kernel(state, t2, out, idxv, rowsv, sem):
    c = lax.axis_index("c")
    s = lax.axis_index("s")

    @pl.when(s == 0)
    def _():
        pltpu.sync_copy(t2.at[pl.ds(c * _NEG, _NEG)], idxv)
        idxv[...] = idxv[...] + c * _N
        pltpu.async_copy(state.at[idxv], rowsv, sem).wait()
        pltpu.sync_copy(rowsv, out.at[pl.ds(c * _NEG, _NEG)])


def _prep_body(r0_ref, h0_ref, qt_ref, q_ref, bd_ref):
    b = pl.program_id(0)
    j = pl.program_id(1)
    oh_r = (lax.broadcasted_iota(jnp.int32, (1, _R), 1) == r0_ref[b]
            ).astype(jnp.float32)
    q = jnp.dot(oh_r, qt_ref[...], preferred_element_type=jnp.float32)
    q_ref[0] = q
    rows = lax.broadcasted_iota(jnp.int32, (_BLKN, 1), 0) + j * _BLKN
    bd_ref[0] = (rows == h0_ref[b]).astype(jnp.float32) * q


_BLKN = 2000


def _prep(r0, h0, query_table):
    return pl.pallas_call(
        _prep_body,
        grid=(_B, _N // _BLKN),
        in_specs=[
            pl.BlockSpec(memory_space=pltpu.SMEM),
            pl.BlockSpec(memory_space=pltpu.SMEM),
            pl.BlockSpec((_R, _D), lambda b, j: (0, 0)),
        ],
        out_specs=[
            pl.BlockSpec((1, 1, _D), lambda b, j: (b, 0, 0)),
            pl.BlockSpec((1, _BLKN, _D), lambda b, j: (b, j, 0)),
        ],
        out_shape=[
            jax.ShapeDtypeStruct((_B, 1, _D), jnp.float32),
            jax.ShapeDtypeStruct((_B, _N, _D), jnp.float32),
        ],
    )(r0, h0, query_table)


def _dense_body(x_ref, agg_ref, bd_ref, wa_ref, wb_ref, b_ref, o_ref):
    xv = x_ref[...]
    a = agg_ref[...] + bd_ref[...]
    h = (jnp.dot(xv, wa_ref[...], preferred_element_type=jnp.float32)
         + jnp.dot(a, wb_ref[...], preferred_element_type=jnp.float32)
         + b_ref[...])
    o_ref[...] = jnp.maximum(h, 0.0) + xv


def _dense(x, agg, bd, wa, wb, b):
    blk = 2000
    rows = _B * _N
    return pl.pallas_call(
        _dense_body,
        grid=(rows // blk,),
        in_specs=[
            pl.BlockSpec((blk, _D), lambda i: (i, 0)),
            pl.BlockSpec((blk, _D), lambda i: (i, 0)),
            pl.BlockSpec((blk, _D), lambda i: (i, 0)),
            pl.BlockSpec((_D, _D), lambda i: (0, 0)),
            pl.BlockSpec((_D, _D), lambda i: (0, 0)),
            pl.BlockSpec((1, _D), lambda i: (0, 0)),
        ],
        out_specs=pl.BlockSpec((blk, _D), lambda i: (i, 0)),
        out_shape=jax.ShapeDtypeStruct((rows, _D), jnp.float32),
    )(x, agg, bd, wa, wb, b)


def _readout_body(g_ref, q_ref, a_ref, bm_ref, b0_ref, w1_ref, b1_ref, o_ref):
    x = (jnp.dot(g_ref[...], a_ref[...], preferred_element_type=jnp.float32)
         + jnp.dot(q_ref[...], bm_ref[...], preferred_element_type=jnp.float32)
         + b0_ref[...])
    x = jnp.maximum(x, 0.0)
    s = jnp.sum(x * w1_ref[...], axis=1, keepdims=True)
    o_ref[...] = s + b1_ref[...]


def _readout(g, qrep, a, bm, b0, w1row, b1row):
    rows = _B * _NEG
    return pl.pallas_call(
        _readout_body,
        out_shape=jax.ShapeDtypeStruct((rows, _D), jnp.float32),
    )(g, qrep, a, bm, b0, w1row, b1row)


def kernel(batch, edge_index, edge_type, query_table, rel0, W0, b0, rel1, W1,
           b1, mlp_W0, mlp_b0, mlp_W1, mlp_b1):
    h_index = batch[..., 0]
    t_index = batch[..., 1]
    r_index = batch[..., 2]
    is_t_neg = jnp.all(h_index == h_index[:, :1], axis=-1, keepdims=True)
    h2 = jnp.where(is_t_neg, h_index, t_index)
    t2 = jnp.where(is_t_neg, t_index, h_index)
    h0 = h2[:, 0]
    r0 = r_index[:, 0]

    pad = _EPAD - _E
    src_p = jnp.concatenate([edge_index[0],
                             jnp.zeros((pad,), jnp.int32)])
    dst_p = jnp.concatenate([edge_index[1],
                             _N + (jnp.arange(pad, dtype=jnp.int32) % 16)])
    et_p = jnp.concatenate([edge_type, jnp.zeros((pad,), jnp.int32)])

    query3, boundary = _prep(r0, h0, query_table)
    query = query3.reshape(_B, _D)
    bdf = boundary.reshape(_B * _N, _D)

    x = bdf
    for rel, W, b in ((rel0, W0, b0), (rel1, W1, b1)):
        agg = _mp_kernel(x, src_p, dst_p, et_p, rel)
        x = _dense(x, agg, bdf, W[:_D], W[_D:], b.reshape(1, _D))

    g = _tail_gather_kernel(x, t2.reshape(-1))
    qrep = jnp.repeat(query, _NEG, axis=0)
    out = _readout(g, qrep, mlp_W0[:_D], mlp_W0[_D:],
                   mlp_b0.reshape(1, 2 * _D), mlp_W1.reshape(1, 2 * _D),
                   jnp.broadcast_to(mlp_b1.reshape(1, 1), (1, _D)))
    return out[:, 0].reshape(_B, _NEG)


# speedup vs baseline: 28.2487x; 1.2686x over previous
import functools

import jax
import jax.numpy as jnp
from jax import lax
from jax.experimental import pallas as pl
from jax.experimental.pallas import tpu as pltpu
from jax.experimental.pallas import tpu_sc as plsc

_N = 10000
_E = 160000
_R = 32
_D = 128
_B = 2
_NEG = 16

_CH = 128
_EPT = 10240
_EPAD = _EPT * 16
_NCHUNK = _EPT // _CH
_EEXTRA = 2 * _CH
_NACC = 10240
_ZROWS = _NACC // 16

_mesh = plsc.VectorSubcoreMesh(core_axis_name="c", subcore_axis_name="s")


@functools.partial(
    pl.kernel,
    mesh=_mesh,
    out_type=jax.ShapeDtypeStruct((_B * _N, _D), jnp.float32),
    scratch_types=[
        pltpu.VMEM((_CH,), jnp.int32),
        pltpu.VMEM((_CH,), jnp.int32),
        pltpu.VMEM((_CH,), jnp.int32),
        pltpu.VMEM((_CH,), jnp.int32),
        pltpu.VMEM((_CH,), jnp.int32),
        pltpu.VMEM((_CH,), jnp.int32),
        pltpu.VMEM((_CH,), jnp.int32),
        pltpu.VMEM((_CH,), jnp.int32),
        pltpu.VMEM((_CH, _D), jnp.float32),
        pltpu.VMEM((_CH, _D), jnp.float32),
        pltpu.VMEM((_R, _D), jnp.float32),
        pltpu.VMEM_SHARED((_NACC, _D), jnp.float32),
        pltpu.SemaphoreType.DMA,
        pltpu.SemaphoreType.DMA,
        pltpu.SemaphoreType.DMA,
        pltpu.SemaphoreType.DMA,
        pltpu.SemaphoreType.DMA,
        pltpu.SemaphoreType.DMA,
    ],
)
def _mp_kernel(state, src, dst, et, rel, out,
               s0, d0, t0, s1, d1, t1, ds0, ds1, r0, r1, relv, acc,
               g0, g1, sc0, sc1, i0, i1):
    c = lax.axis_index("c")
    s = lax.axis_index("s")
    S = (s0, s1)
    Dd = (d0, d1)
    T = (t0, t1)
    DS = (ds0, ds1)
    RW = (r0, r1)
    G = (g0, g1)
    SC = (sc0, sc1)
    I = (i0, i1)

    pltpu.sync_copy(rel, relv)

    zero16 = jnp.zeros((16,), jnp.float32)

    def _zrow(i, carry):
        for j in range(_D // 16):
            r0[i, pl.ds(j * 16, 16)] = zero16
        return carry

    lax.fori_loop(0, _CH, _zrow, 0)
    zbase = s * _ZROWS
    for k in range(_ZROWS // _CH):
        pltpu.sync_copy(r0, acc.at[pl.ds(zbase + k * _CH, _CH)])
    plsc.subcore_barrier()

    ebase = s * _EPT
    shift = c * _N

    def _issue_idx(b, k):
        off = ebase + k * _CH
        pltpu.async_copy(src.at[pl.ds(off, _CH)], S[b], I[b])
        pltpu.async_copy(dst.at[pl.ds(off, _CH)], Dd[b], I[b])
        pltpu.async_copy(et.at[pl.ds(off, _CH)], T[b], I[b])

    def _drain_idx(b):
        for ref in (S[b], Dd[b], T[b]):
            pltpu.make_async_copy(src.at[pl.ds(0, _CH)], ref, I[b]).wait()

    def _shift_src(b):
        for v in range(_CH // 16):
            S[b][pl.ds(v * 16, 16)] = S[b][pl.ds(v * 16, 16)] + shift

    def _issue_gather(b):
        pltpu.async_copy(state.at[S[b]], RW[b], G[b])

    def _drain_gather(b):
        pltpu.make_async_copy(state.at[pl.ds(0, _CH)], RW[b], G[b]).wait()

    def _drain_scatter(b):
        pltpu.make_async_copy(state.at[pl.ds(0, _CH)],
                              acc.at[pl.ds(_N, _CH)], SC[b]).wait()

    def _multiply(b):
        rows = RW[b]
        etv = T[b]

        def _group(g, ecarry):
            et_vec = etv[pl.ds(g * 16, 16)]
            for l in range(16):
                et_i = et_vec[l]
                i = g * 16 + l
                for j in range(_D // 16):
                    rows[i, pl.ds(j * 16, 16)] = (
                        rows[i, pl.ds(j * 16, 16)]
                        * relv[et_i, pl.ds(j * 16, 16)])
            return ecarry

        lax.fori_loop(0, _CH // 16, _group, 0)

    _issue_idx(0, 0)
    _drain_idx(0)
    _shift_src(0)
    _issue_gather(0)
    _issue_idx(1, 1)

    def _pipeline(j, carry):
        k2 = 2 * j
        for b in (0, 1):
            k = k2 + b
            q = 1 - b
            _drain_idx(q)
            _shift_src(q)
            _issue_gather(q)
            _drain_gather(b)
            _multiply(b)
            pltpu.sync_copy(RW[b], acc.at[Dd[b]], add=True)
            _issue_idx(b, k + 2)
        return carry

    lax.fori_loop(0, _NCHUNK // 2, _pipeline, 0)
    _drain_gather(0)
    _drain_idx(1)
    plsc.subcore_barrier()

    pltpu.sync_copy(acc.at[pl.ds(s * 624, 624)],
                    out.at[pl.ds(c * _N + s * 624, 624)])

    @pl.when(s == 0)
    def _():
        pltpu.sync_copy(acc.at[pl.ds(9984, 16)],
                        out.at[pl.ds(c * _N + 9984, 16)])


@functools.partial(
    pl.kernel,
    mesh=_mesh,
    out_type=jax.ShapeDtypeStruct((_B * _NEG, _D), jnp.float32),
    scratch_types=[
        pltpu.VMEM((_NEG,), jnp.int32),
        pltpu.VMEM((_NEG, _D), jnp.float32),
        pltpu.SemaphoreType.DMA,
    ],
)
def _tail_gather_kernel(state, t2, out, idxv, rowsv, sem):
    c = lax.axis_index("c")
    s = lax.axis_index("s")

    @pl.when(s == 0)
    def _():
        pltpu.sync_copy(t2.at[pl.ds(c * _NEG, _NEG)], idxv)
        idxv[...] = idxv[...] + c * _N
        pltpu.async_copy(state.at[idxv], rowsv, sem).wait()
        pltpu.sync_copy(rowsv, out.at[pl.ds(c * _NEG, _NEG)])


def _prep_body(r0_ref, h0_ref, qt_ref, q_ref, bd_ref):
    b = pl.program_id(0)
    j = pl.program_id(1)
    oh_r = (lax.broadcasted_iota(jnp.int32, (1, _R), 1) == r0_ref[b]
            ).astype(jnp.float32)
    q = jnp.dot(oh_r, qt_ref[...], preferred_element_type=jnp.float32)
    q_ref[0] = q
    rows = lax.broadcasted_iota(jnp.int32, (_BLKN, 1), 0) + j * _BLKN
    bd_ref[0] = (rows == h0_ref[b]).astype(jnp.float32) * q


_BLKN = 2000


def _prep(r0, h0, query_table):
    return pl.pallas_call(
        _prep_body,
        grid=(_B, _N // _BLKN),
        in_specs=[
            pl.BlockSpec(memory_space=pltpu.SMEM),
            pl.BlockSpec(memory_space=pltpu.SMEM),
            pl.BlockSpec((_R, _D), lambda b, j: (0, 0)),
        ],
        out_specs=[
            pl.BlockSpec((1, 1, _D), lambda b, j: (b, 0, 0)),
            pl.BlockSpec((1, _BLKN, _D), lambda b, j: (b, j, 0)),
        ],
        out_shape=[
            jax.ShapeDtypeStruct((_B, 1, _D), jnp.float32),
            jax.ShapeDtypeStruct((_B, _N, _D), jnp.float32),
        ],
    )(r0, h0, query_table)


def _dense_body(x_ref, agg_ref, bd_ref, wa_ref, wb_ref, b_ref, o_ref):
    xv = x_ref[...]
    a = agg_ref[...] + bd_ref[...]
    h = (jnp.dot(xv, wa_ref[...], preferred_element_type=jnp.float32)
         + jnp.dot(a, wb_ref[...], preferred_element_type=jnp.float32)
         + b_ref[...])
    o_ref[...] = jnp.maximum(h, 0.0) + xv


def _dense(x, agg, bd, wa, wb, b):
    blk = 2000
    rows = _B * _N
    return pl.pallas_call(
        _dense_body,
        grid=(rows // blk,),
        in_specs=[
            pl.BlockSpec((blk, _D), lambda i: (i, 0)),
            pl.BlockSpec((blk, _D), lambda i: (i, 0)),
            pl.BlockSpec((blk, _D), lambda i: (i, 0)),
            pl.BlockSpec((_D, _D), lambda i: (0, 0)),
            pl.BlockSpec((_D, _D), lambda i: (0, 0)),
            pl.BlockSpec((1, _D), lambda i: (0, 0)),
        ],
        out_specs=pl.BlockSpec((blk, _D), lambda i: (i, 0)),
        out_shape=jax.ShapeDtypeStruct((rows, _D), jnp.float32),
    )(x, agg, bd, wa, wb, b)


def _readout_body(g_ref, q_ref, a_ref, bm_ref, b0_ref, w1_ref, b1_ref, o_ref):
    x = (jnp.dot(g_ref[...], a_ref[...], preferred_element_type=jnp.float32)
         + jnp.dot(q_ref[...], bm_ref[...], preferred_element_type=jnp.float32)
         + b0_ref[...])
    x = jnp.maximum(x, 0.0)
    s = jnp.sum(x * w1_ref[...], axis=1, keepdims=True)
    o_ref[...] = s + b1_ref[...]


def _readout(g, qrep, a, bm, b0, w1row, b1row):
    rows = _B * _NEG
    return pl.pallas_call(
        _readout_body,
        out_shape=jax.ShapeDtypeStruct((rows, _D), jnp.float32),
    )(g, qrep, a, bm, b0, w1row, b1row)


def kernel(batch, edge_index, edge_type, query_table, rel0, W0, b0, rel1, W1,
           b1, mlp_W0, mlp_b0, mlp_W1, mlp_b1):
    h_index = batch[..., 0]
    t_index = batch[..., 1]
    r_index = batch[..., 2]
    is_t_neg = jnp.all(h_index == h_index[:, :1], axis=-1, keepdims=True)
    h2 = jnp.where(is_t_neg, h_index, t_index)
    t2 = jnp.where(is_t_neg, t_index, h_index)
    h0 = h2[:, 0]
    r0 = r_index[:, 0]

    pad = _EPAD + _EEXTRA - _E
    src_p = jnp.concatenate([edge_index[0],
                             jnp.zeros((pad,), jnp.int32)])
    dst_p = jnp.concatenate([edge_index[1],
                             _N + (jnp.arange(pad, dtype=jnp.int32) % 16)])
    et_p = jnp.concatenate([edge_type, jnp.zeros((pad,), jnp.int32)])

    query3, boundary = _prep(r0, h0, query_table)
    query = query3.reshape(_B, _D)
    bdf = boundary.reshape(_B * _N, _D)

    x = bdf
    for rel, W, b in ((rel0, W0, b0), (rel1, W1, b1)):
        agg = _mp_kernel(x, src_p, dst_p, et_p, rel)
        x = _dense(x, agg, bdf, W[:_D], W[_D:], b.reshape(1, _D))

    g = _tail_gather_kernel(x, t2.reshape(-1))
    qrep = jnp.repeat(query, _NEG, axis=0)
    out = _readout(g, qrep, mlp_W0[:_D], mlp_W0[_D:],
                   mlp_b0.reshape(1, 2 * _D), mlp_W1.reshape(1, 2 * _D),
                   jnp.broadcast_to(mlp_b1.reshape(1, 1), (1, _D)))
    return out[:, 0].reshape(_B, _NEG)


# speedup vs baseline: 53.4856x; 1.8934x over previous
import functools

import jax
import jax.numpy as jnp
from jax import lax
from jax.experimental import pallas as pl
from jax.experimental.pallas import tpu as pltpu
from jax.experimental.pallas import tpu_sc as plsc

_N = 10000
_E = 160000
_R = 32
_D = 128
_B = 2
_NEG = 16

_CH = 128
_EPT = 10240
_EPAD = _EPT * 16
_NCHUNK = _EPT // _CH
_EEXTRA = 2 * _CH
_NACC = 10240
_ZROWS = _NACC // 16

_mesh = plsc.VectorSubcoreMesh(core_axis_name="c", subcore_axis_name="s")


@functools.partial(
    pl.kernel,
    mesh=_mesh,
    out_type=jax.ShapeDtypeStruct((_B * _N, _D), jnp.float32),
    scratch_types=[
        pltpu.VMEM((_CH,), jnp.int32),
        pltpu.VMEM((_CH,), jnp.int32),
        pltpu.VMEM((_CH,), jnp.int32),
        pltpu.VMEM((_CH,), jnp.int32),
        pltpu.VMEM((_CH,), jnp.int32),
        pltpu.VMEM((_CH,), jnp.int32),
        pltpu.VMEM((_CH,), jnp.int32),
        pltpu.VMEM((_CH,), jnp.int32),
        pltpu.VMEM((_CH, _D), jnp.float32),
        pltpu.VMEM((_CH, _D), jnp.float32),
        pltpu.VMEM((_R, _D), jnp.float32),
        pltpu.VMEM_SHARED((_NACC, _D), jnp.float32),
        pltpu.SemaphoreType.DMA,
        pltpu.SemaphoreType.DMA,
        pltpu.SemaphoreType.DMA,
        pltpu.SemaphoreType.DMA,
        pltpu.SemaphoreType.DMA,
        pltpu.SemaphoreType.DMA,
    ],
)
def _mp_kernel(state, src, dst, et, rel, out,
               s0, d0, t0, s1, d1, t1, ds0, ds1, r0, r1, relv, acc,
               g0, g1, sc0, sc1, i0, i1):
    c = lax.axis_index("c")
    s = lax.axis_index("s")
    S = (s0, s1)
    Dd = (d0, d1)
    T = (t0, t1)
    DS = (ds0, ds1)
    RW = (r0, r1)
    G = (g0, g1)
    SC = (sc0, sc1)
    I = (i0, i1)

    pltpu.sync_copy(rel, relv)

    zero16 = jnp.zeros((16,), jnp.float32)

    def _zrow(i, carry):
        for j in range(_D // 16):
            r0[i, pl.ds(j * 16, 16)] = zero16
        return carry

    lax.fori_loop(0, _CH, _zrow, 0)
    zbase = s * _ZROWS
    for k in range(_ZROWS // _CH):
        pltpu.sync_copy(r0, acc.at[pl.ds(zbase + k * _CH, _CH)])
    plsc.subcore_barrier()

    ebase = s * _EPT
    shift = c * _N

    def _issue_idx(b, k):
        off = ebase + k * _CH
        pltpu.async_copy(src.at[pl.ds(off, _CH)], S[b], I[b])
        pltpu.async_copy(dst.at[pl.ds(off, _CH)], Dd[b], I[b])
        pltpu.async_copy(et.at[pl.ds(off, _CH)], T[b], I[b])

    def _drain_idx(b):
        for ref in (S[b], Dd[b], T[b]):
            pltpu.make_async_copy(src.at[pl.ds(0, _CH)], ref, I[b]).wait()

    def _shift_src(b):
        for v in range(_CH // 16):
            S[b][pl.ds(v * 16, 16)] = S[b][pl.ds(v * 16, 16)] + shift

    def _issue_gather(b):
        pltpu.async_copy(state.at[S[b]], RW[b], G[b])

    def _drain_gather(b):
        pltpu.make_async_copy(state.at[pl.ds(0, _CH)], RW[b], G[b]).wait()

    def _drain_scatter(b):
        pltpu.make_async_copy(state.at[pl.ds(0, _CH)],
                              acc.at[pl.ds(_N, _CH)], SC[b]).wait()

    def _multiply(b):
        rows = RW[b]
        etv = T[b]

        def _group(g, ecarry):
            et_vec = etv[pl.ds(g * 16, 16)]
            for l in range(16):
                et_i = et_vec[l]
                i = g * 16 + l
                for j in range(_D // 16):
                    rows[i, pl.ds(j * 16, 16)] = (
                        rows[i, pl.ds(j * 16, 16)]
                        * relv[et_i, pl.ds(j * 16, 16)])
            return ecarry

        lax.fori_loop(0, _CH // 16, _group, 0)

    _issue_idx(0, 0)
    _drain_idx(0)
    _shift_src(0)
    _issue_gather(0)
    _issue_idx(1, 1)

    def _pipeline(j, carry):
        k2 = 2 * j
        for b in (0, 1):
            k = k2 + b
            q = 1 - b
            _drain_idx(q)
            _shift_src(q)
            _issue_gather(q)
            _drain_gather(b)
            _multiply(b)
            pltpu.sync_copy(RW[b], acc.at[Dd[b]], add=True)
            _issue_idx(b, k + 2)
        return carry

    lax.fori_loop(0, _NCHUNK // 2, _pipeline, 0)
    _drain_gather(0)
    _drain_idx(1)
    plsc.subcore_barrier()

    pltpu.sync_copy(acc.at[pl.ds(s * 624, 624)],
                    out.at[pl.ds(c * _N + s * 624, 624)])

    @pl.when(s == 0)
    def _():
        pltpu.sync_copy(acc.at[pl.ds(9984, 16)],
                        out.at[pl.ds(c * _N + 9984, 16)])


_SCH = 1024


@functools.partial(
    pl.kernel,
    mesh=_mesh,
    out_type=jax.ShapeDtypeStruct((_B * _N, _D), jnp.float32),
    scratch_types=[
        pltpu.VMEM((_SCH,), jnp.int32),
        pltpu.VMEM((_SCH,), jnp.int32),
        pltpu.VMEM((_SCH,), jnp.int32),
        pltpu.VMEM((_CH,), jnp.int32),
        pltpu.VMEM((_CH, _D), jnp.float32),
        pltpu.VMEM((_D,), jnp.float32),
        pltpu.VMEM((16,), jnp.int32),
        pltpu.VMEM((_R, _D), jnp.float32),
        pltpu.VMEM_SHARED((_NACC, _D), jnp.float32),
        pltpu.SemaphoreType.DMA,
    ],
)
def _mp0_kernel(qflat, h0p, src, dst, et, rel, out,
                srcb, dstb, etb, dsb, rows, qv, h0v, relv, acc, sem):
    c = lax.axis_index("c")
    s = lax.axis_index("s")

    pltpu.sync_copy(qflat.at[pl.ds(c * _D, _D)], qv)
    pltpu.sync_copy(h0p, h0v)
    pltpu.sync_copy(rel, relv)
    hv = h0v[...]
    h0s = jnp.where(c == 0, hv[0], hv[1])

    qregs = [qv[pl.ds(j * 16, 16)] for j in range(_D // 16)]

    def _qr(r, carry):
        for j in range(_D // 16):
            relv[r, pl.ds(j * 16, 16)] = relv[r, pl.ds(j * 16, 16)] * qregs[j]
        return carry

    lax.fori_loop(0, _R, _qr, 0)

    zero16 = jnp.zeros((16,), jnp.float32)

    def _zrow(i, carry):
        for j in range(_D // 16):
            rows[i, pl.ds(j * 16, 16)] = zero16
        return carry

    lax.fori_loop(0, _CH, _zrow, 0)
    zbase = s * _ZROWS
    for k in range(_ZROWS // _CH):
        pltpu.sync_copy(rows, acc.at[pl.ds(zbase + k * _CH, _CH)])
    plsc.subcore_barrier()

    ebase = s * _EPT

    def _scan(k, carry):
        off = ebase + k * _SCH
        h1 = pltpu.async_copy(src.at[pl.ds(off, _SCH)], srcb, sem)
        h2 = pltpu.async_copy(dst.at[pl.ds(off, _SCH)], dstb, sem)
        h3 = pltpu.async_copy(et.at[pl.ds(off, _SCH)], etb, sem)
        h1.wait()
        h2.wait()
        h3.wait()

        def _sub(s2, carry2):
            base = s2 * _CH
            cv = jnp.zeros((16,), jnp.int32)
            for g in range(_CH // 16):
                sv = srcb[pl.ds(base + g * 16, 16)]
                cv = cv + jnp.where(sv == h0s, 1, 0)
            cnt = cv[0]
            for l in range(1, 16):
                cnt = cnt + cv[l]

            @pl.when(cnt > 0)
            def _():
                for v in range(_CH // 16):
                    dsb[pl.ds(v * 16, 16)] = dstb[pl.ds(base + v * 16, 16)]

                def _fill(g, c3):
                    sv = srcb[pl.ds(base + g * 16, 16)]
                    ev = etb[pl.ds(base + g * 16, 16)]
                    for l in range(16):
                        @pl.when(sv[l] == h0s)
                        def _():
                            i = g * 16 + l
                            for j in range(_D // 16):
                                rows[i, pl.ds(j * 16, 16)] = (
                                    relv[ev[l], pl.ds(j * 16, 16)])
                    return c3

                lax.fori_loop(0, _CH // 16, _fill, 0)
                pltpu.sync_copy(rows, acc.at[dsb], add=True)

                def _rezero(g, c3):
                    sv = srcb[pl.ds(base + g * 16, 16)]
                    for l in range(16):
                        @pl.when(sv[l] == h0s)
                        def _():
                            i = g * 16 + l
                            for j in range(_D // 16):
                                rows[i, pl.ds(j * 16, 16)] = zero16
                    return c3

                lax.fori_loop(0, _CH // 16, _rezero, 0)

            return carry2

        return lax.fori_loop(0, _SCH // _CH, _sub, carry)

    lax.fori_loop(0, _EPT // _SCH, _scan, 0)
    plsc.subcore_barrier()

    pltpu.sync_copy(acc.at[pl.ds(s * 624, 624)],
                    out.at[pl.ds(c * _N + s * 624, 624)])

    @pl.when(s == 0)
    def _():
        pltpu.sync_copy(acc.at[pl.ds(9984, 16)],
                        out.at[pl.ds(c * _N + 9984, 16)])


@functools.partial(
    pl.kernel,
    mesh=_mesh,
    out_type=jax.ShapeDtypeStruct((_B * _NEG, _D), jnp.float32),
    scratch_types=[
        pltpu.VMEM((_NEG,), jnp.int32),
        pltpu.VMEM((_NEG, _D), jnp.float32),
        pltpu.SemaphoreType.DMA,
    ],
)
def _tail_gather_kernel(state, t2, out, idxv, rowsv, sem):
    c = lax.axis_index("c")
    s = lax.axis_index("s")

    @pl.when(s == 0)
    def _():
        pltpu.sync_copy(t2.at[pl.ds(c * _NEG, _NEG)], idxv)
        idxv[...] = idxv[...] + c * _N
        pltpu.async_copy(state.at[idxv], rowsv, sem).wait()
        pltpu.sync_copy(rowsv, out.at[pl.ds(c * _NEG, _NEG)])


def _prep_body(r0_ref, h0_ref, qt_ref, q_ref, bd_ref):
    b = pl.program_id(0)
    j = pl.program_id(1)
    oh_r = (lax.broadcasted_iota(jnp.int32, (1, _R), 1) == r0_ref[b]
            ).astype(jnp.float32)
    q = jnp.dot(oh_r, qt_ref[...], preferred_element_type=jnp.float32)
    q_ref[0] = q
    rows = lax.broadcasted_iota(jnp.int32, (_BLKN, 1), 0) + j * _BLKN
    bd_ref[0] = (rows == h0_ref[b]).astype(jnp.float32) * q


_BLKN = 2000


def _prep(r0, h0, query_table):
    return pl.pallas_call(
        _prep_body,
        grid=(_B, _N // _BLKN),
        in_specs=[
            pl.BlockSpec(memory_space=pltpu.SMEM),
            pl.BlockSpec(memory_space=pltpu.SMEM),
            pl.BlockSpec((_R, _D), lambda b, j: (0, 0)),
        ],
        out_specs=[
            pl.BlockSpec((1, 1, _D), lambda b, j: (b, 0, 0)),
            pl.BlockSpec((1, _BLKN, _D), lambda b, j: (b, j, 0)),
        ],
        out_shape=[
            jax.ShapeDtypeStruct((_B, 1, _D), jnp.float32),
            jax.ShapeDtypeStruct((_B, _N, _D), jnp.float32),
        ],
    )(r0, h0, query_table)


def _dense_body(x_ref, agg_ref, bd_ref, wa_ref, wb_ref, b_ref, o_ref):
    xv = x_ref[...]
    a = agg_ref[...] + bd_ref[...]
    h = (jnp.dot(xv, wa_ref[...], preferred_element_type=jnp.float32)
         + jnp.dot(a, wb_ref[...], preferred_element_type=jnp.float32)
         + b_ref[...])
    o_ref[...] = jnp.maximum(h, 0.0) + xv


def _dense(x, agg, bd, wa, wb, b):
    blk = 2000
    rows = _B * _N
    return pl.pallas_call(
        _dense_body,
        grid=(rows // blk,),
        in_specs=[
            pl.BlockSpec((blk, _D), lambda i: (i, 0)),
            pl.BlockSpec((blk, _D), lambda i: (i, 0)),
            pl.BlockSpec((blk, _D), lambda i: (i, 0)),
            pl.BlockSpec((_D, _D), lambda i: (0, 0)),
            pl.BlockSpec((_D, _D), lambda i: (0, 0)),
            pl.BlockSpec((1, _D), lambda i: (0, 0)),
        ],
        out_specs=pl.BlockSpec((blk, _D), lambda i: (i, 0)),
        out_shape=jax.ShapeDtypeStruct((rows, _D), jnp.float32),
    )(x, agg, bd, wa, wb, b)


def _readout_body(g_ref, q_ref, a_ref, bm_ref, b0_ref, w1_ref, b1_ref, o_ref):
    x = (jnp.dot(g_ref[...], a_ref[...], preferred_element_type=jnp.float32)
         + jnp.dot(q_ref[...], bm_ref[...], preferred_element_type=jnp.float32)
         + b0_ref[...])
    x = jnp.maximum(x, 0.0)
    s = jnp.sum(x * w1_ref[...], axis=1, keepdims=True)
    o_ref[...] = s + b1_ref[...]


def _readout(g, qrep, a, bm, b0, w1row, b1row):
    rows = _B * _NEG
    return pl.pallas_call(
        _readout_body,
        out_shape=jax.ShapeDtypeStruct((rows, _D), jnp.float32),
    )(g, qrep, a, bm, b0, w1row, b1row)


def kernel(batch, edge_index, edge_type, query_table, rel0, W0, b0, rel1, W1,
           b1, mlp_W0, mlp_b0, mlp_W1, mlp_b1):
    h_index = batch[..., 0]
    t_index = batch[..., 1]
    r_index = batch[..., 2]
    is_t_neg = jnp.all(h_index == h_index[:, :1], axis=-1, keepdims=True)
    h2 = jnp.where(is_t_neg, h_index, t_index)
    t2 = jnp.where(is_t_neg, t_index, h_index)
    h0 = h2[:, 0]
    r0 = r_index[:, 0]

    pad = _EPAD + _EEXTRA - _E
    src_p = jnp.concatenate([edge_index[0],
                             jnp.zeros((pad,), jnp.int32)])
    dst_p = jnp.concatenate([edge_index[1],
                             _N + (jnp.arange(pad, dtype=jnp.int32) % 16)])
    et_p = jnp.concatenate([edge_type, jnp.zeros((pad,), jnp.int32)])

    query3, boundary = _prep(r0, h0, query_table)
    query = query3.reshape(_B, _D)
    bdf = boundary.reshape(_B * _N, _D)

    qflat = query.reshape(-1)
    h0p = jnp.concatenate([h0, jnp.zeros((16 - _B,), jnp.int32)])
    agg0 = _mp0_kernel(qflat, h0p, src_p, dst_p, et_p, rel0)
    x = _dense(bdf, agg0, bdf, W0[:_D], W0[_D:], b0.reshape(1, _D))
    agg1 = _mp_kernel(x, src_p, dst_p, et_p, rel1)
    x = _dense(x, agg1, bdf, W1[:_D], W1[_D:], b1.reshape(1, _D))

    g = _tail_gather_kernel(x, t2.reshape(-1))
    qrep = jnp.repeat(query, _NEG, axis=0)
    out = _readout(g, qrep, mlp_W0[:_D], mlp_W0[_D:],
                   mlp_b0.reshape(1, 2 * _D), mlp_W1.reshape(1, 2 * _D),
                   jnp.broadcast_to(mlp_b1.reshape(1, 1), (1, _D)))
    return out[:, 0].reshape(_B, _NEG)


# speedup vs baseline: 53.8135x; 1.0061x over previous
import functools

import jax
import jax.numpy as jnp
from jax import lax
from jax.experimental import pallas as pl
from jax.experimental.pallas import tpu as pltpu
from jax.experimental.pallas import tpu_sc as plsc

_N = 10000
_E = 160000
_R = 32
_D = 128
_B = 2
_NEG = 16

_CH = 128
_EPT = 10240
_EPAD = _EPT * 16
_NCHUNK = _EPT // _CH
_EEXTRA = 2 * _CH
_NACC = 10240
_ZROWS = _NACC // 16

_mesh = plsc.VectorSubcoreMesh(core_axis_name="c", subcore_axis_name="s")


@functools.partial(
    pl.kernel,
    mesh=_mesh,
    out_type=jax.ShapeDtypeStruct((_B * _N, _D), jnp.float32),
    scratch_types=[
        pltpu.VMEM((_CH,), jnp.int32),
        pltpu.VMEM((_CH,), jnp.int32),
        pltpu.VMEM((_CH,), jnp.int32),
        pltpu.VMEM((_CH,), jnp.int32),
        pltpu.VMEM((_CH,), jnp.int32),
        pltpu.VMEM((_CH,), jnp.int32),
        pltpu.VMEM((_CH,), jnp.int32),
        pltpu.VMEM((_CH,), jnp.int32),
        pltpu.VMEM((_CH, _D), jnp.float32),
        pltpu.VMEM((_CH, _D), jnp.float32),
        pltpu.VMEM((_R, _D), jnp.float32),
        pltpu.VMEM_SHARED((_NACC, _D), jnp.float32),
        pltpu.SemaphoreType.DMA,
        pltpu.SemaphoreType.DMA,
        pltpu.SemaphoreType.DMA,
        pltpu.SemaphoreType.DMA,
        pltpu.SemaphoreType.DMA,
        pltpu.SemaphoreType.DMA,
    ],
)
def _mp_kernel(state, src, dst, et, rel, out,
               s0, d0, t0, s1, d1, t1, ds0, ds1, r0, r1, relv, acc,
               g0, g1, sc0, sc1, i0, i1):
    c = lax.axis_index("c")
    s = lax.axis_index("s")
    S = (s0, s1)
    Dd = (d0, d1)
    T = (t0, t1)
    DS = (ds0, ds1)
    RW = (r0, r1)
    G = (g0, g1)
    SC = (sc0, sc1)
    I = (i0, i1)

    pltpu.sync_copy(rel, relv)

    zero16 = jnp.zeros((16,), jnp.float32)

    def _zrow(i, carry):
        for j in range(_D // 16):
            r0[i, pl.ds(j * 16, 16)] = zero16
        return carry

    lax.fori_loop(0, _CH, _zrow, 0)
    zbase = s * _ZROWS
    for k in range(_ZROWS // _CH):
        pltpu.sync_copy(r0, acc.at[pl.ds(zbase + k * _CH, _CH)])
    plsc.subcore_barrier()

    ebase = s * _EPT
    shift = c * _N

    def _issue_idx(b, k):
        off = ebase + k * _CH
        pltpu.async_copy(src.at[pl.ds(off, _CH)], S[b], I[b])
        pltpu.async_copy(dst.at[pl.ds(off, _CH)], Dd[b], I[b])
        pltpu.async_copy(et.at[pl.ds(off, _CH)], T[b], I[b])

    def _drain_idx(b):
        for ref in (S[b], Dd[b], T[b]):
            pltpu.make_async_copy(src.at[pl.ds(0, _CH)], ref, I[b]).wait()

    def _shift_src(b):
        for v in range(_CH // 16):
            S[b][pl.ds(v * 16, 16)] = S[b][pl.ds(v * 16, 16)] + shift

    def _issue_gather(b):
        pltpu.async_copy(state.at[S[b]], RW[b], G[b])

    def _drain_gather(b):
        pltpu.make_async_copy(state.at[pl.ds(0, _CH)], RW[b], G[b]).wait()

    def _drain_scatter(b):
        pltpu.make_async_copy(state.at[pl.ds(0, _CH)],
                              acc.at[pl.ds(_N, _CH)], SC[b]).wait()

    def _multiply(b):
        rows = RW[b]
        etv = T[b]

        def _group(g, ecarry):
            et_vec = etv[pl.ds(g * 16, 16)]
            for l in range(16):
                et_i = et_vec[l]
                i = g * 16 + l
                for j in range(_D // 16):
                    rows[i, pl.ds(j * 16, 16)] = (
                        rows[i, pl.ds(j * 16, 16)]
                        * relv[et_i, pl.ds(j * 16, 16)])
            return ecarry

        lax.fori_loop(0, _CH // 16, _group, 0, unroll=2)

    _issue_idx(0, 0)
    _drain_idx(0)
    _shift_src(0)
    _issue_gather(0)
    _issue_idx(1, 1)

    def _pipeline(j, carry):
        k2 = 2 * j
        for b in (0, 1):
            k = k2 + b
            q = 1 - b
            _drain_idx(q)
            _shift_src(q)
            _issue_gather(q)
            _drain_gather(b)
            _multiply(b)
            pltpu.sync_copy(RW[b], acc.at[Dd[b]], add=True)
            _issue_idx(b, k + 2)
        return carry

    lax.fori_loop(0, _NCHUNK // 2, _pipeline, 0)
    _drain_gather(0)
    _drain_idx(1)
    plsc.subcore_barrier()

    pltpu.sync_copy(acc.at[pl.ds(s * 624, 624)],
                    out.at[pl.ds(c * _N + s * 624, 624)])

    @pl.when(s == 0)
    def _():
        pltpu.sync_copy(acc.at[pl.ds(9984, 16)],
                        out.at[pl.ds(c * _N + 9984, 16)])


_SCH = 1024


@functools.partial(
    pl.kernel,
    mesh=_mesh,
    out_type=jax.ShapeDtypeStruct((_B * _N, _D), jnp.float32),
    scratch_types=[
        pltpu.VMEM((_SCH,), jnp.int32),
        pltpu.VMEM((_SCH,), jnp.int32),
        pltpu.VMEM((_SCH,), jnp.int32),
        pltpu.VMEM((_CH,), jnp.int32),
        pltpu.VMEM((_CH, _D), jnp.float32),
        pltpu.VMEM((_D,), jnp.float32),
        pltpu.VMEM((16,), jnp.int32),
        pltpu.VMEM((_R, _D), jnp.float32),
        pltpu.VMEM_SHARED((_NACC, _D), jnp.float32),
        pltpu.SemaphoreType.DMA,
    ],
)
def _mp0_kernel(qflat, h0p, src, dst, et, rel, out,
                srcb, dstb, etb, dsb, rows, qv, h0v, relv, acc, sem):
    c = lax.axis_index("c")
    s = lax.axis_index("s")

    pltpu.sync_copy(qflat.at[pl.ds(c * _D, _D)], qv)
    pltpu.sync_copy(h0p, h0v)
    pltpu.sync_copy(rel, relv)
    hv = h0v[...]
    h0s = jnp.where(c == 0, hv[0], hv[1])

    qregs = [qv[pl.ds(j * 16, 16)] for j in range(_D // 16)]

    def _qr(r, carry):
        for j in range(_D // 16):
            relv[r, pl.ds(j * 16, 16)] = relv[r, pl.ds(j * 16, 16)] * qregs[j]
        return carry

    lax.fori_loop(0, _R, _qr, 0)

    zero16 = jnp.zeros((16,), jnp.float32)

    def _zrow(i, carry):
        for j in range(_D // 16):
            rows[i, pl.ds(j * 16, 16)] = zero16
        return carry

    lax.fori_loop(0, _CH, _zrow, 0)
    zbase = s * _ZROWS
    for k in range(_ZROWS // _CH):
        pltpu.sync_copy(rows, acc.at[pl.ds(zbase + k * _CH, _CH)])
    plsc.subcore_barrier()

    ebase = s * _EPT

    def _scan(k, carry):
        off = ebase + k * _SCH
        h1 = pltpu.async_copy(src.at[pl.ds(off, _SCH)], srcb, sem)
        h2 = pltpu.async_copy(dst.at[pl.ds(off, _SCH)], dstb, sem)
        h3 = pltpu.async_copy(et.at[pl.ds(off, _SCH)], etb, sem)
        h1.wait()
        h2.wait()
        h3.wait()

        def _sub(s2, carry2):
            base = s2 * _CH
            cv = jnp.zeros((16,), jnp.int32)
            for g in range(_CH // 16):
                sv = srcb[pl.ds(base + g * 16, 16)]
                cv = cv + jnp.where(sv == h0s, 1, 0)
            cnt = cv[0]
            for l in range(1, 16):
                cnt = cnt + cv[l]

            @pl.when(cnt > 0)
            def _():
                for v in range(_CH // 16):
                    dsb[pl.ds(v * 16, 16)] = dstb[pl.ds(base + v * 16, 16)]

                def _fill(g, c3):
                    sv = srcb[pl.ds(base + g * 16, 16)]
                    ev = etb[pl.ds(base + g * 16, 16)]
                    for l in range(16):
                        @pl.when(sv[l] == h0s)
                        def _():
                            i = g * 16 + l
                            for j in range(_D // 16):
                                rows[i, pl.ds(j * 16, 16)] = (
                                    relv[ev[l], pl.ds(j * 16, 16)])
                    return c3

                lax.fori_loop(0, _CH // 16, _fill, 0)
                pltpu.sync_copy(rows, acc.at[dsb], add=True)

                def _rezero(g, c3):
                    sv = srcb[pl.ds(base + g * 16, 16)]
                    for l in range(16):
                        @pl.when(sv[l] == h0s)
                        def _():
                            i = g * 16 + l
                            for j in range(_D // 16):
                                rows[i, pl.ds(j * 16, 16)] = zero16
                    return c3

                lax.fori_loop(0, _CH // 16, _rezero, 0)

            return carry2

        return lax.fori_loop(0, _SCH // _CH, _sub, carry)

    lax.fori_loop(0, _EPT // _SCH, _scan, 0)
    plsc.subcore_barrier()

    pltpu.sync_copy(acc.at[pl.ds(s * 624, 624)],
                    out.at[pl.ds(c * _N + s * 624, 624)])

    @pl.when(s == 0)
    def _():
        pltpu.sync_copy(acc.at[pl.ds(9984, 16)],
                        out.at[pl.ds(c * _N + 9984, 16)])


@functools.partial(
    pl.kernel,
    mesh=_mesh,
    out_type=jax.ShapeDtypeStruct((_B * _NEG, _D), jnp.float32),
    scratch_types=[
        pltpu.VMEM((_NEG,), jnp.int32),
        pltpu.VMEM((_NEG, _D), jnp.float32),
        pltpu.SemaphoreType.DMA,
    ],
)
def _tail_gather_kernel(state, t2, out, idxv, rowsv, sem):
    c = lax.axis_index("c")
    s = lax.axis_index("s")

    @pl.when(s == 0)
    def _():
        pltpu.sync_copy(t2.at[pl.ds(c * _NEG, _NEG)], idxv)
        idxv[...] = idxv[...] + c * _N
        pltpu.async_copy(state.at[idxv], rowsv, sem).wait()
        pltpu.sync_copy(rowsv, out.at[pl.ds(c * _NEG, _NEG)])


def _prep_body(r0_ref, h0_ref, qt_ref, q_ref, bd_ref):
    b = pl.program_id(0)
    j = pl.program_id(1)
    oh_r = (lax.broadcasted_iota(jnp.int32, (1, _R), 1) == r0_ref[b]
            ).astype(jnp.float32)
    q = jnp.dot(oh_r, qt_ref[...], preferred_element_type=jnp.float32)
    q_ref[0] = q
    rows = lax.broadcasted_iota(jnp.int32, (_BLKN, 1), 0) + j * _BLKN
    bd_ref[0] = (rows == h0_ref[b]).astype(jnp.float32) * q


_BLKN = 2000


def _prep(r0, h0, query_table):
    return pl.pallas_call(
        _prep_body,
        grid=(_B, _N // _BLKN),
        in_specs=[
            pl.BlockSpec(memory_space=pltpu.SMEM),
            pl.BlockSpec(memory_space=pltpu.SMEM),
            pl.BlockSpec((_R, _D), lambda b, j: (0, 0)),
        ],
        out_specs=[
            pl.BlockSpec((1, 1, _D), lambda b, j: (b, 0, 0)),
            pl.BlockSpec((1, _BLKN, _D), lambda b, j: (b, j, 0)),
        ],
        out_shape=[
            jax.ShapeDtypeStruct((_B, 1, _D), jnp.float32),
            jax.ShapeDtypeStruct((_B, _N, _D), jnp.float32),
        ],
    )(r0, h0, query_table)


def _dense_body(x_ref, agg_ref, bd_ref, wa_ref, wb_ref, b_ref, o_ref):
    xv = x_ref[...]
    a = agg_ref[...] + bd_ref[...]
    h = (jnp.dot(xv, wa_ref[...], preferred_element_type=jnp.float32)
         + jnp.dot(a, wb_ref[...], preferred_element_type=jnp.float32)
         + b_ref[...])
    o_ref[...] = jnp.maximum(h, 0.0) + xv


def _dense(x, agg, bd, wa, wb, b):
    blk = 2000
    rows = _B * _N
    return pl.pallas_call(
        _dense_body,
        grid=(rows // blk,),
        in_specs=[
            pl.BlockSpec((blk, _D), lambda i: (i, 0)),
            pl.BlockSpec((blk, _D), lambda i: (i, 0)),
            pl.BlockSpec((blk, _D), lambda i: (i, 0)),
            pl.BlockSpec((_D, _D), lambda i: (0, 0)),
            pl.BlockSpec((_D, _D), lambda i: (0, 0)),
            pl.BlockSpec((1, _D), lambda i: (0, 0)),
        ],
        out_specs=pl.BlockSpec((blk, _D), lambda i: (i, 0)),
        out_shape=jax.ShapeDtypeStruct((rows, _D), jnp.float32),
    )(x, agg, bd, wa, wb, b)


def _readout_body(g_ref, q_ref, a_ref, bm_ref, b0_ref, w1_ref, b1_ref, o_ref):
    x = (jnp.dot(g_ref[...], a_ref[...], preferred_element_type=jnp.float32)
         + jnp.dot(q_ref[...], bm_ref[...], preferred_element_type=jnp.float32)
         + b0_ref[...])
    x = jnp.maximum(x, 0.0)
    s = jnp.sum(x * w1_ref[...], axis=1, keepdims=True)
    o_ref[...] = s + b1_ref[...]


def _readout(g, qrep, a, bm, b0, w1row, b1row):
    rows = _B * _NEG
    return pl.pallas_call(
        _readout_body,
        out_shape=jax.ShapeDtypeStruct((rows, _D), jnp.float32),
    )(g, qrep, a, bm, b0, w1row, b1row)


def kernel(batch, edge_index, edge_type, query_table, rel0, W0, b0, rel1, W1,
           b1, mlp_W0, mlp_b0, mlp_W1, mlp_b1):
    h_index = batch[..., 0]
    t_index = batch[..., 1]
    r_index = batch[..., 2]
    is_t_neg = jnp.all(h_index == h_index[:, :1], axis=-1, keepdims=True)
    h2 = jnp.where(is_t_neg, h_index, t_index)
    t2 = jnp.where(is_t_neg, t_index, h_index)
    h0 = h2[:, 0]
    r0 = r_index[:, 0]

    pad = _EPAD + _EEXTRA - _E
    src_p = jnp.concatenate([edge_index[0],
                             jnp.zeros((pad,), jnp.int32)])
    dst_p = jnp.concatenate([edge_index[1],
                             _N + (jnp.arange(pad, dtype=jnp.int32) % 16)])
    et_p = jnp.concatenate([edge_type, jnp.zeros((pad,), jnp.int32)])

    query3, boundary = _prep(r0, h0, query_table)
    query = query3.reshape(_B, _D)
    bdf = boundary.reshape(_B * _N, _D)

    qflat = query.reshape(-1)
    h0p = jnp.concatenate([h0, jnp.zeros((16 - _B,), jnp.int32)])
    agg0 = _mp0_kernel(qflat, h0p, src_p, dst_p, et_p, rel0)
    x = _dense(bdf, agg0, bdf, W0[:_D], W0[_D:], b0.reshape(1, _D))
    agg1 = _mp_kernel(x, src_p, dst_p, et_p, rel1)
    x = _dense(x, agg1, bdf, W1[:_D], W1[_D:], b1.reshape(1, _D))

    g = _tail_gather_kernel(x, t2.reshape(-1))
    qrep = jnp.repeat(query, _NEG, axis=0)
    out = _readout(g, qrep, mlp_W0[:_D], mlp_W0[_D:],
                   mlp_b0.reshape(1, 2 * _D), mlp_W1.reshape(1, 2 * _D),
                   jnp.broadcast_to(mlp_b1.reshape(1, 1), (1, _D)))
    return out[:, 0].reshape(_B, _NEG)
